# baseline (speedup 1.0000x reference)
import functools

import jax
import jax.numpy as jnp
import numpy as _np
from jax import lax
from jax.experimental import pallas as pl
from jax.experimental.pallas import tpu as pltpu
from jax.experimental.pallas import tpu_sc as plsc

_F = 128
_HALF = _F // 2
_NRBF = 20
_CUT = 5.0
_CGCUT = 20.0
_NA = 10000
_NCG = 1000
_E = 320000

_BPAD = 32
_AT = 1000
_NAT = _NA // _AT
_ET2 = 2560
_NET2 = _E // _ET2
_NW = 32
_EPW = _E // _NW
_CH = 16
_NCHUNK = _EPW // _CH
_ZR = 8
_GPT = (_NA // _ZR) // 16
_U = 5
_NIT = _NCHUNK // _U


def _swish(x):
    return x * (1.0 / (1.0 + jnp.exp(-x)))


def _mlp(h, w1, b1, w2, b2):
    t = jnp.dot(h, w1, preferred_element_type=jnp.float32) + b1
    return jnp.dot(_swish(t), w2, preferred_element_type=jnp.float32) + b2


def _rbfenv(d, cutoff, rows):
    n = (lax.broadcasted_iota(jnp.int32, (1, _NRBF), 1) + 1
         ).astype(jnp.float32) * (_np.pi / cutoff)
    rbf = jnp.sin(d * n) / d
    env = jnp.where(d < cutoff, 0.5 * (jnp.cos(d * (jnp.pi / cutoff)) + 1.0), 0.0)
    pad = jnp.zeros((rows, _BPAD - _NRBF - 1), jnp.float32)
    return jnp.concatenate([rbf * env, env, pad], axis=1)


def _k0_body(z_ref, map_ref, cgz_ref, xyz_ref, cgxyz_ref, at_ref, rt_ref,
             w1_ref, b1_ref, w2_ref, b2_ref,
             h_ref, phi_ref, b_ref, cnt_ref):
    a = pl.program_id(0)
    zb = z_ref[0, 0, :]
    mb = map_ref[0, 0, :]
    cgz = cgz_ref[0, :]
    oh_z = (zb[:, None] == lax.broadcasted_iota(jnp.int32, (_AT, 100), 1)
            ).astype(jnp.float32)
    h_at = jnp.dot(oh_z, at_ref[...], preferred_element_type=jnp.float32)
    oh_cgz = (cgz[:, None] == lax.broadcasted_iota(jnp.int32, (_NCG, 30), 1)
              ).astype(jnp.float32)
    res_cg = jnp.dot(oh_cgz, rt_ref[...], preferred_element_type=jnp.float32)
    oh_m = (mb[:, None] == lax.broadcasted_iota(jnp.int32, (_AT, _NCG), 1)
            ).astype(jnp.float32)
    h_res = jnp.dot(oh_m, res_cg, preferred_element_type=jnp.float32)
    h = jnp.concatenate([h_at, h_res], axis=1)
    h_ref[...] = h
    phi_ref[...] = _mlp(h, w1_ref[...], b1_ref[...], w2_ref[...], b2_ref[...])
    cg_g = jnp.dot(oh_m, cgxyz_ref[...], preferred_element_type=jnp.float32)
    r = xyz_ref[...] - cg_g
    d = jnp.sqrt(jnp.sum(r * r, axis=1, keepdims=True))
    b_ref[...] = _rbfenv(d, _CGCUT, _AT)

    @pl.when(a == 0)
    def _():
        cnt_ref[...] = jnp.zeros_like(cnt_ref)

    cnt_ref[...] += jnp.sum(oh_m, axis=0)[:, None]


def _wf_body(r_ref, wd_ref, o0_ref, o1_ref, o2_ref, bscr):
    r = r_ref[...]
    d2c = jnp.sum(r * r, axis=1, keepdims=True)
    d2 = d2c.reshape(1, _ET2)
    d = jnp.sqrt(d2)
    x = d * (_np.pi / _CUT)
    s1 = jnp.sin(x)
    c1 = jnp.cos(x)
    env = jnp.where(d < _CUT, 0.5 * (c1 + 1.0), 0.0)
    g = env / d
    twoc = c1 + c1
    bscr[0:1, :] = s1 * g
    s_prev, s_cur = s1, twoc * s1
    bscr[1:2, :] = s_cur * g
    for k in range(2, _NRBF):
        s_nxt = twoc * s_cur - s_prev
        bscr[k:k + 1, :] = s_nxt * g
        s_prev, s_cur = s_cur, s_nxt
    bscr[_NRBF:_NRBF + 1, :] = env
    bscr[_NRBF + 1:_BPAD, :] = jnp.zeros((_BPAD - _NRBF - 1, _ET2), jnp.float32)
    w = lax.dot_general(bscr[...], wd_ref[...], (((0,), (0,)), ((), ())),
                        preferred_element_type=jnp.float32)
    o0_ref[...] = w[:, :_F]
    o1_ref[...] = w[:, _F:2 * _F]
    o2_ref[...] = w[:, 2 * _F:]


def _c1a_body(last, *refs):
    if last:
        h_ref, osc_ref, hout_ref = refs
    else:
        h_ref, osc_ref, mw1, mb1, mw2, mb2, hout_ref, phin_ref = refs
    hn = h_ref[...] + osc_ref[0] + osc_ref[1]
    hout_ref[...] = hn
    if not last:
        phin_ref[...] = _mlp(hn, mw1[...], mb1[...], mw2[...], mb2[...])


def _c1b_body(first, *refs):
    (hn_ref, map_ref, b_ref, cnt_ref, hin_ref,
     cw1, cb1, cw2, cb2, cwd, hcg_ref, accA, accX) = refs
    a = pl.program_id(0)
    hn = hn_ref[...]
    phic = _mlp(hn, cw1[...], cb1[...], cw2[...], cb2[...])
    x = phic * jnp.dot(b_ref[...], cwd[...], preferred_element_type=jnp.float32)
    mb = map_ref[0, 0, :]
    oh_t = (mb[None, :] == lax.broadcasted_iota(jnp.int32, (_NCG, _AT), 0)
            ).astype(jnp.float32)

    @pl.when(a == 0)
    def _():
        accX[...] = jnp.zeros_like(accX)
        if first:
            accA[...] = jnp.zeros_like(accA)

    accX[...] += jnp.dot(oh_t, x, preferred_element_type=jnp.float32)
    if first:
        accA[...] += jnp.dot(oh_t, hn, preferred_element_type=jnp.float32)

    @pl.when(a == _NAT - 1)
    def _():
        hnew = hin_ref[...] + accX[...]
        if first:
            hnew = hnew + accA[...] / jnp.maximum(cnt_ref[...], 1.0)
        hcg_ref[...] = hnew


_sc_mesh = plsc.VectorSubcoreMesh(core_axis_name="c", subcore_axis_name="s")


def _ke_body(xyz_ref, src_ref, dst_ref, out_ref, srcv, dstv, gs, gd, ob,
             *sems):
    c = lax.axis_index("c")
    s = lax.axis_index("s")
    wid = s * 2 + c
    base = wid * _EPW
    pltpu.sync_copy(src_ref.at[pl.ds(base, _EPW)], srcv)
    pltpu.sync_copy(dst_ref.at[pl.ds(base, _EPW)], dstv)

    def body(it, carry):
        e0 = it * (_U * _CH)
        cps = []
        for k in range(_U):
            iv = srcv[pl.ds(e0 + k * _CH, _CH)]
            idv = dstv[pl.ds(e0 + k * _CH, _CH)]
            cps.append(pltpu.async_copy(
                xyz_ref.at[iv], gs.at[pl.ds(k * _CH, _CH)], sems[2 * k]))
            cps.append(pltpu.async_copy(
                xyz_ref.at[idv], gd.at[pl.ds(k * _CH, _CH)], sems[2 * k + 1]))
        for k in range(_U):
            cps[2 * k].wait()
            cps[2 * k + 1].wait()
            for j in range(_CH):
                r = k * _CH + j
                ob[r, :] = gs[r, :] - gd[r, :]
        pltpu.sync_copy(ob, out_ref.at[pl.ds(base + e0, _U * _CH)])
        return carry

    lax.fori_loop(0, _NIT, body, 0)


_ke = pl.kernel(
    _ke_body,
    out_type=jax.ShapeDtypeStruct((_E, 16), jnp.float32),
    mesh=_sc_mesh,
    scratch_types=[
        pltpu.VMEM((_EPW,), jnp.int32),
        pltpu.VMEM((_EPW,), jnp.int32),
        pltpu.VMEM((_U * _CH, 16), jnp.float32),
        pltpu.VMEM((_U * _CH, 16), jnp.float32),
        pltpu.VMEM((_U * _CH, 16), jnp.float32),
    ] + [pltpu.SemaphoreType.DMA] * (2 * _U),
    compiler_params=pltpu.CompilerParams(use_tc_tiling_on_sc=False),
)


def _scb_body(phi_ref, w_ref, src_ref, dst_ref, out_ref,
              srcv, dstv, gb, wb, zb, acc, *sems):
    c = lax.axis_index("c")
    s = lax.axis_index("s")
    wid = s * 2 + c
    base = wid * _EPW
    row0 = s * (_GPT * _ZR)
    ngroups = jnp.where(s == 15, _NA // _ZR - 15 * _GPT, _GPT)
    zeros16 = jnp.zeros((16,), jnp.float32)
    for r in range(_ZR):
        for q in range(8):
            zb[r, pl.ds(q * 16, 16)] = zeros16

    def zbody(k, carry):
        pltpu.sync_copy(zb, acc.at[pl.ds(row0 + k * _ZR, _ZR)])
        return carry

    lax.fori_loop(0, ngroups, zbody, 0)
    pltpu.sync_copy(src_ref.at[pl.ds(base, _EPW)], srcv)
    pltpu.sync_copy(dst_ref.at[pl.ds(base, _EPW)], dstv)
    plsc.subcore_barrier()

    def body(it, carry):
        e0 = it * (_U * _CH)
        cps = []
        for k in range(_U):
            iv = srcv[pl.ds(e0 + k * _CH, _CH)]
            cps.append(pltpu.async_copy(
                phi_ref.at[iv], gb.at[pl.ds(k * _CH, _CH)], sems[k]))
        pltpu.sync_copy(w_ref.at[pl.ds(base + e0, _U * _CH)], wb)
        for k in range(_U):
            cps[k].wait()
            for j in range(_CH):
                r = k * _CH + j
                for q in range(8):
                    sl = pl.ds(q * 16, 16)
                    gb[r, sl] = gb[r, sl] * wb[r, sl]
            idv = dstv[pl.ds(e0 + k * _CH, _CH)]
            pltpu.sync_copy(gb.at[pl.ds(k * _CH, _CH)], acc.at[idv], add=True)
        return carry

    lax.fori_loop(0, _NIT, body, 0)
    plsc.subcore_barrier()

    def cbody(k, carry):
        sl = pl.ds(row0 + k * _ZR, _ZR)
        pltpu.sync_copy(acc.at[sl], out_ref.at[c, sl])
        return carry

    lax.fori_loop(0, ngroups, cbody, 0)


_scb = pl.kernel(
    _scb_body,
    out_type=jax.ShapeDtypeStruct((2, _NA, _F), jnp.float32),
    mesh=_sc_mesh,
    scratch_types=[
        pltpu.VMEM((_EPW,), jnp.int32),
        pltpu.VMEM((_EPW,), jnp.int32),
        pltpu.VMEM((_U * _CH, _F), jnp.float32),
        pltpu.VMEM((_U * _CH, _F), jnp.float32),
        pltpu.VMEM((_ZR, _F), jnp.float32),
        pltpu.VMEM_SHARED((_NA, _F), jnp.float32),
    ] + [pltpu.SemaphoreType.DMA] * _U,
)


def _full(spec):
    return pl.BlockSpec(spec, lambda a: tuple(0 for _ in spec))


def kernel(z, xyz, cg_z, cg_xyz, mapping, nbr_list, cg_nbr_list, ic,
           atom_table, res_table,
           msg_W1, msg_b1, msg_W2, msg_b2, msg_Wd, msg_bd,
           cg_W1, cg_b1, cg_W2, cg_b2, cg_Wd, cg_bd):
    f32 = jnp.float32
    z3 = z.astype(jnp.int32).reshape(_NAT, 1, _AT)
    map3 = mapping.astype(jnp.int32).reshape(_NAT, 1, _AT)
    cgz2 = cg_z.astype(jnp.int32).reshape(1, _NCG)
    src = nbr_list[:, 1].astype(jnp.int32)
    dst = nbr_list[:, 0].astype(jnp.int32)
    xyz16 = jnp.concatenate([xyz, jnp.zeros((_NA, 13), f32)], axis=1)

    def aug(wd, bd):
        w = jnp.concatenate([wd[:, _F:2 * _F], bd[None, _F:2 * _F]], axis=0)
        return jnp.concatenate([w, jnp.zeros((_BPAD - _NRBF - 1, _F), f32)], 0)

    m_wd = [aug(msg_Wd[i], msg_bd[i]) for i in range(3)]
    c_wd = [aug(cg_Wd[i], cg_bd[i]) for i in range(3)]
    m_w2 = [msg_W2[i][:, _F:2 * _F] for i in range(3)]
    m_b2 = [msg_b2[i][None, _F:2 * _F] for i in range(3)]
    c_w2 = [cg_W2[i][:, _F:2 * _F] for i in range(3)]
    c_b2 = [cg_b2[i][None, _F:2 * _F] for i in range(3)]
    m_b1 = [msg_b1[i][None, :] for i in range(3)]
    c_b1 = [cg_b1[i][None, :] for i in range(3)]

    blk_a = pl.BlockSpec((_AT, _F), lambda a: (a, 0))
    blk_i3 = pl.BlockSpec((1, 1, _AT), lambda a: (a, 0, 0))

    h, phi, b_ii, cnt = pl.pallas_call(
        _k0_body,
        grid=(_NAT,),
        in_specs=[
            blk_i3, blk_i3, _full((1, _NCG)),
            pl.BlockSpec((_AT, 3), lambda a: (a, 0)),
            _full((_NCG, 3)), _full((100, _HALF)), _full((30, _HALF)),
            _full((_F, _F)), _full((1, _F)), _full((_F, _F)), _full((1, _F)),
        ],
        out_specs=[
            blk_a, blk_a,
            pl.BlockSpec((_AT, _BPAD), lambda a: (a, 0)),
            _full((_NCG, 1)),
        ],
        out_shape=[
            jax.ShapeDtypeStruct((_NA, _F), f32),
            jax.ShapeDtypeStruct((_NA, _F), f32),
            jax.ShapeDtypeStruct((_NA, _BPAD), f32),
            jax.ShapeDtypeStruct((_NCG, 1), f32),
        ],
    )(z3, map3, cgz2, xyz, cg_xyz, atom_table, res_table,
      msg_W1[0], m_b1[0], m_w2[0], m_b2[0])

    r16 = _ke(xyz16, src, dst)

    wd_all = jnp.concatenate(m_wd, axis=1)
    blk_e = pl.BlockSpec((_ET2, _F), lambda a: (a, 0))
    w_list = pl.pallas_call(
        _wf_body,
        grid=(_NET2,),
        in_specs=[pl.BlockSpec((_ET2, 16), lambda a: (a, 0)),
                  _full((_BPAD, 3 * _F))],
        out_specs=[blk_e, blk_e, blk_e],
        out_shape=[jax.ShapeDtypeStruct((_E, _F), f32)] * 3,
        scratch_shapes=[pltpu.VMEM((_BPAD, _ET2), f32)],
    )(r16, wd_all)

    def c1a_call(i, h, osc):
        last = i == 2
        if last:
            return pl.pallas_call(
                functools.partial(_c1a_body, True),
                grid=(_NAT,),
                in_specs=[blk_a, pl.BlockSpec((2, _AT, _F), lambda a: (0, a, 0))],
                out_specs=blk_a,
                out_shape=jax.ShapeDtypeStruct((_NA, _F), f32),
            )(h, osc), None
        nxt = i + 1
        return pl.pallas_call(
            functools.partial(_c1a_body, False),
            grid=(_NAT,),
            in_specs=[
                blk_a, pl.BlockSpec((2, _AT, _F), lambda a: (0, a, 0)),
                _full((_F, _F)), _full((1, _F)), _full((_F, _F)),
                _full((1, _F)),
            ],
            out_specs=[blk_a, blk_a],
            out_shape=[jax.ShapeDtypeStruct((_NA, _F), f32),
                       jax.ShapeDtypeStruct((_NA, _F), f32)],
        )(h, osc, msg_W1[nxt], m_b1[nxt], m_w2[nxt], m_b2[nxt])

    def c1b_call(i, hn, big_h):
        return pl.pallas_call(
            functools.partial(_c1b_body, i == 0),
            grid=(_NAT,),
            in_specs=[
                blk_a, blk_i3,
                pl.BlockSpec((_AT, _BPAD), lambda a: (a, 0)),
                _full((_NCG, 1)), _full((_NCG, _F)),
                _full((_F, _F)), _full((1, _F)), _full((_F, _F)),
                _full((1, _F)), _full((_BPAD, _F)),
            ],
            out_specs=_full((_NCG, _F)),
            out_shape=jax.ShapeDtypeStruct((_NCG, _F), f32),
            scratch_shapes=[pltpu.VMEM((_NCG, _F), f32),
                            pltpu.VMEM((_NCG, _F), f32)],
        )(hn, map3, b_ii, cnt, big_h,
          cg_W1[i], c_b1[i], c_w2[i], c_b2[i], c_wd[i])

    big_h = jnp.zeros((_NCG, _F), f32)
    for i in range(3):
        osc = _scb(phi, w_list[i], src, dst)
        if i > 0:
            big_h = c1b_call(i - 1, h, big_h)
        hn, phi = c1a_call(i, h, osc)
        h = hn
    big_h = c1b_call(2, h, big_h)

    return (big_h, h)


# speedup vs baseline: 1.7277x; 1.7277x over previous
import functools

import jax
import jax.numpy as jnp
import numpy as _np
from jax import lax
from jax.experimental import pallas as pl
from jax.experimental.pallas import tpu as pltpu
from jax.experimental.pallas import tpu_sc as plsc

_F = 128
_HALF = _F // 2
_NRBF = 20
_CUT = 5.0
_CGCUT = 20.0
_NA = 10000
_NCG = 1000
_E = 320000

_BPAD = 32
_AT = 1000
_NAT = _NA // _AT
_ET2 = 4000
_NET2 = _E // _ET2
_NW = 32
_EPW = _E // _NW
_CH = 16
_NCHUNK = _EPW // _CH
_ZR = 8
_GPT = (_NA // _ZR) // 16
_U = 5
_NIT = _NCHUNK // _U


def _swish(x):
    return x * (1.0 / (1.0 + jnp.exp(-x)))


def _mlp(h, w1, b1, w2, b2):
    t = jnp.dot(h, w1, preferred_element_type=jnp.float32) + b1
    return jnp.dot(_swish(t), w2, preferred_element_type=jnp.float32) + b2


def _rbfenv(d, cutoff, rows):
    n = (lax.broadcasted_iota(jnp.int32, (1, _NRBF), 1) + 1
         ).astype(jnp.float32) * (_np.pi / cutoff)
    rbf = jnp.sin(d * n) / d
    env = jnp.where(d < cutoff, 0.5 * (jnp.cos(d * (jnp.pi / cutoff)) + 1.0), 0.0)
    pad = jnp.zeros((rows, _BPAD - _NRBF - 1), jnp.float32)
    return jnp.concatenate([rbf * env, env, pad], axis=1)


def _k0_body(z_ref, map_ref, cgz_ref, xyz_ref, cgxyz_ref, at_ref, rt_ref,
             w1_ref, b1_ref, w2_ref, b2_ref,
             h_ref, phi_ref, b_ref, cnt_ref):
    a = pl.program_id(0)
    zb = z_ref[0, 0, :]
    mb = map_ref[0, 0, :]
    cgz = cgz_ref[0, :]
    oh_z = (zb[:, None] == lax.broadcasted_iota(jnp.int32, (_AT, 100), 1)
            ).astype(jnp.float32)
    h_at = jnp.dot(oh_z, at_ref[...], preferred_element_type=jnp.float32)
    oh_cgz = (cgz[:, None] == lax.broadcasted_iota(jnp.int32, (_NCG, 30), 1)
              ).astype(jnp.float32)
    res_cg = jnp.dot(oh_cgz, rt_ref[...], preferred_element_type=jnp.float32)
    oh_m = (mb[:, None] == lax.broadcasted_iota(jnp.int32, (_AT, _NCG), 1)
            ).astype(jnp.float32)
    h_res = jnp.dot(oh_m, res_cg, preferred_element_type=jnp.float32)
    h = jnp.concatenate([h_at, h_res], axis=1)
    h_ref[...] = h
    phi_ref[...] = _mlp(h, w1_ref[...], b1_ref[...], w2_ref[...], b2_ref[...])
    cg_g = jnp.dot(oh_m, cgxyz_ref[...], preferred_element_type=jnp.float32)
    r = xyz_ref[...] - cg_g
    d = jnp.sqrt(jnp.sum(r * r, axis=1, keepdims=True))
    b_ref[...] = _rbfenv(d, _CGCUT, _AT)

    @pl.when(a == 0)
    def _():
        cnt_ref[...] = jnp.zeros_like(cnt_ref)

    cnt_ref[...] += jnp.sum(oh_m, axis=0)[:, None]


def _wf_body(r_ref, wd_ref, o0_ref, o1_ref, o2_ref):
    r = r_ref[...]
    d2 = r[:, 0:1] * r[:, 0:1] + r[:, 1:2] * r[:, 1:2] + r[:, 2:3] * r[:, 2:3]
    b = _rbfenv(jnp.sqrt(d2), _CUT, _ET2)
    w = jnp.dot(b, wd_ref[...], preferred_element_type=jnp.float32)
    o0_ref[...] = w[:, :_F]
    o1_ref[...] = w[:, _F:2 * _F]
    o2_ref[...] = w[:, 2 * _F:]


def _c1a_body(last, *refs):
    if last:
        h_ref, osc_ref, hout_ref = refs
    else:
        h_ref, osc_ref, mw1, mb1, mw2, mb2, hout_ref, phin_ref = refs
    hn = h_ref[...] + osc_ref[0] + osc_ref[1]
    hout_ref[...] = hn
    if not last:
        phin_ref[...] = _mlp(hn, mw1[...], mb1[...], mw2[...], mb2[...])


def _c1b_body(first, *refs):
    (hn_ref, map_ref, b_ref, cnt_ref, hin_ref,
     cw1, cb1, cw2, cb2, cwd, hcg_ref, accA, accX) = refs
    a = pl.program_id(0)
    hn = hn_ref[...]
    phic = _mlp(hn, cw1[...], cb1[...], cw2[...], cb2[...])
    x = phic * jnp.dot(b_ref[...], cwd[...], preferred_element_type=jnp.float32)
    mb = map_ref[0, 0, :]
    oh_t = (mb[None, :] == lax.broadcasted_iota(jnp.int32, (_NCG, _AT), 0)
            ).astype(jnp.float32)

    @pl.when(a == 0)
    def _():
        accX[...] = jnp.zeros_like(accX)
        if first:
            accA[...] = jnp.zeros_like(accA)

    accX[...] += jnp.dot(oh_t, x, preferred_element_type=jnp.float32)
    if first:
        accA[...] += jnp.dot(oh_t, hn, preferred_element_type=jnp.float32)

    @pl.when(a == _NAT - 1)
    def _():
        hnew = hin_ref[...] + accX[...]
        if first:
            hnew = hnew + accA[...] / jnp.maximum(cnt_ref[...], 1.0)
        hcg_ref[...] = hnew


_sc_mesh = plsc.VectorSubcoreMesh(core_axis_name="c", subcore_axis_name="s")


def _ke_body(xyz_ref, src_ref, dst_ref, out_ref, srcv, dstv, gs, gd, ob,
             *sems):
    c = lax.axis_index("c")
    s = lax.axis_index("s")
    wid = s * 2 + c
    base = wid * _EPW
    pltpu.sync_copy(src_ref.at[pl.ds(base, _EPW)], srcv)
    pltpu.sync_copy(dst_ref.at[pl.ds(base, _EPW)], dstv)

    def body(it, carry):
        e0 = it * (_U * _CH)
        cps = []
        for k in range(_U):
            iv = srcv[pl.ds(e0 + k * _CH, _CH)]
            idv = dstv[pl.ds(e0 + k * _CH, _CH)]
            cps.append(pltpu.async_copy(
                xyz_ref.at[iv], gs.at[pl.ds(k * _CH, _CH)], sems[2 * k]))
            cps.append(pltpu.async_copy(
                xyz_ref.at[idv], gd.at[pl.ds(k * _CH, _CH)], sems[2 * k + 1]))
        for k in range(_U):
            cps[2 * k].wait()
            cps[2 * k + 1].wait()
            for j in range(_CH):
                r = k * _CH + j
                ob[r, :] = gs[r, :] - gd[r, :]
        pltpu.sync_copy(ob, out_ref.at[pl.ds(base + e0, _U * _CH)])
        return carry

    lax.fori_loop(0, _NIT, body, 0)


_ke = pl.kernel(
    _ke_body,
    out_type=jax.ShapeDtypeStruct((_E, 16), jnp.float32),
    mesh=_sc_mesh,
    scratch_types=[
        pltpu.VMEM((_EPW,), jnp.int32),
        pltpu.VMEM((_EPW,), jnp.int32),
        pltpu.VMEM((_U * _CH, 16), jnp.float32),
        pltpu.VMEM((_U * _CH, 16), jnp.float32),
        pltpu.VMEM((_U * _CH, 16), jnp.float32),
    ] + [pltpu.SemaphoreType.DMA] * (2 * _U),
    compiler_params=pltpu.CompilerParams(use_tc_tiling_on_sc=False),
)


def _scb_body(phi_ref, w_ref, src_ref, dst_ref, out_ref,
              srcv, dstv, gb, wb, zb, acc, *sems):
    c = lax.axis_index("c")
    s = lax.axis_index("s")
    wid = s * 2 + c
    base = wid * _EPW
    row0 = s * (_GPT * _ZR)
    ngroups = jnp.where(s == 15, _NA // _ZR - 15 * _GPT, _GPT)
    zeros16 = jnp.zeros((16,), jnp.float32)
    for r in range(_ZR):
        for q in range(8):
            zb[r, pl.ds(q * 16, 16)] = zeros16

    def zbody(k, carry):
        pltpu.sync_copy(zb, acc.at[pl.ds(row0 + k * _ZR, _ZR)])
        return carry

    lax.fori_loop(0, ngroups, zbody, 0)
    pltpu.sync_copy(src_ref.at[pl.ds(base, _EPW)], srcv)
    pltpu.sync_copy(dst_ref.at[pl.ds(base, _EPW)], dstv)
    plsc.subcore_barrier()

    def body(it, carry):
        e0 = it * (_U * _CH)
        cps = []
        for k in range(_U):
            iv = srcv[pl.ds(e0 + k * _CH, _CH)]
            cps.append(pltpu.async_copy(
                phi_ref.at[iv], gb.at[pl.ds(k * _CH, _CH)], sems[k]))
        pltpu.sync_copy(w_ref.at[pl.ds(base + e0, _U * _CH)], wb)
        for k in range(_U):
            cps[k].wait()
            for j in range(_CH):
                r = k * _CH + j
                for q in range(8):
                    sl = pl.ds(q * 16, 16)
                    gb[r, sl] = gb[r, sl] * wb[r, sl]
            idv = dstv[pl.ds(e0 + k * _CH, _CH)]
            pltpu.sync_copy(gb.at[pl.ds(k * _CH, _CH)], acc.at[idv], add=True)
        return carry

    lax.fori_loop(0, _NIT, body, 0)
    plsc.subcore_barrier()

    def cbody(k, carry):
        sl = pl.ds(row0 + k * _ZR, _ZR)
        pltpu.sync_copy(acc.at[sl], out_ref.at[c, sl])
        return carry

    lax.fori_loop(0, ngroups, cbody, 0)


_scb = pl.kernel(
    _scb_body,
    out_type=jax.ShapeDtypeStruct((2, _NA, _F), jnp.float32),
    mesh=_sc_mesh,
    scratch_types=[
        pltpu.VMEM((_EPW,), jnp.int32),
        pltpu.VMEM((_EPW,), jnp.int32),
        pltpu.VMEM((_U * _CH, _F), jnp.float32),
        pltpu.VMEM((_U * _CH, _F), jnp.float32),
        pltpu.VMEM((_ZR, _F), jnp.float32),
        pltpu.VMEM_SHARED((_NA, _F), jnp.float32),
    ] + [pltpu.SemaphoreType.DMA] * _U,
)


def _full(spec):
    return pl.BlockSpec(spec, lambda a: tuple(0 for _ in spec))


def kernel(z, xyz, cg_z, cg_xyz, mapping, nbr_list, cg_nbr_list, ic,
           atom_table, res_table,
           msg_W1, msg_b1, msg_W2, msg_b2, msg_Wd, msg_bd,
           cg_W1, cg_b1, cg_W2, cg_b2, cg_Wd, cg_bd):
    f32 = jnp.float32
    z3 = z.astype(jnp.int32).reshape(_NAT, 1, _AT)
    map3 = mapping.astype(jnp.int32).reshape(_NAT, 1, _AT)
    cgz2 = cg_z.astype(jnp.int32).reshape(1, _NCG)
    src = nbr_list[:, 1].astype(jnp.int32)
    dst = nbr_list[:, 0].astype(jnp.int32)
    xyz16 = jnp.concatenate([xyz, jnp.zeros((_NA, 13), f32)], axis=1)

    def aug(wd, bd):
        w = jnp.concatenate([wd[:, _F:2 * _F], bd[None, _F:2 * _F]], axis=0)
        return jnp.concatenate([w, jnp.zeros((_BPAD - _NRBF - 1, _F), f32)], 0)

    m_wd = [aug(msg_Wd[i], msg_bd[i]) for i in range(3)]
    c_wd = [aug(cg_Wd[i], cg_bd[i]) for i in range(3)]
    m_w2 = [msg_W2[i][:, _F:2 * _F] for i in range(3)]
    m_b2 = [msg_b2[i][None, _F:2 * _F] for i in range(3)]
    c_w2 = [cg_W2[i][:, _F:2 * _F] for i in range(3)]
    c_b2 = [cg_b2[i][None, _F:2 * _F] for i in range(3)]
    m_b1 = [msg_b1[i][None, :] for i in range(3)]
    c_b1 = [cg_b1[i][None, :] for i in range(3)]

    blk_a = pl.BlockSpec((_AT, _F), lambda a: (a, 0))
    blk_i3 = pl.BlockSpec((1, 1, _AT), lambda a: (a, 0, 0))

    h, phi, b_ii, cnt = pl.pallas_call(
        _k0_body,
        grid=(_NAT,),
        in_specs=[
            blk_i3, blk_i3, _full((1, _NCG)),
            pl.BlockSpec((_AT, 3), lambda a: (a, 0)),
            _full((_NCG, 3)), _full((100, _HALF)), _full((30, _HALF)),
            _full((_F, _F)), _full((1, _F)), _full((_F, _F)), _full((1, _F)),
        ],
        out_specs=[
            blk_a, blk_a,
            pl.BlockSpec((_AT, _BPAD), lambda a: (a, 0)),
            _full((_NCG, 1)),
        ],
        out_shape=[
            jax.ShapeDtypeStruct((_NA, _F), f32),
            jax.ShapeDtypeStruct((_NA, _F), f32),
            jax.ShapeDtypeStruct((_NA, _BPAD), f32),
            jax.ShapeDtypeStruct((_NCG, 1), f32),
        ],
    )(z3, map3, cgz2, xyz, cg_xyz, atom_table, res_table,
      msg_W1[0], m_b1[0], m_w2[0], m_b2[0])

    r16 = _ke(xyz16, src, dst)

    wd_all = jnp.concatenate(m_wd, axis=1)
    blk_e = pl.BlockSpec((_ET2, _F), lambda a: (a, 0))
    w_list = pl.pallas_call(
        _wf_body,
        grid=(_NET2,),
        in_specs=[pl.BlockSpec((_ET2, 16), lambda a: (a, 0)),
                  _full((_BPAD, 3 * _F))],
        out_specs=[blk_e, blk_e, blk_e],
        out_shape=[jax.ShapeDtypeStruct((_E, _F), f32)] * 3,
    )(r16, wd_all)

    def c1a_call(i, h, osc):
        last = i == 2
        if last:
            return pl.pallas_call(
                functools.partial(_c1a_body, True),
                grid=(_NAT,),
                in_specs=[blk_a, pl.BlockSpec((2, _AT, _F), lambda a: (0, a, 0))],
                out_specs=blk_a,
                out_shape=jax.ShapeDtypeStruct((_NA, _F), f32),
            )(h, osc), None
        nxt = i + 1
        return pl.pallas_call(
            functools.partial(_c1a_body, False),
            grid=(_NAT,),
            in_specs=[
                blk_a, pl.BlockSpec((2, _AT, _F), lambda a: (0, a, 0)),
                _full((_F, _F)), _full((1, _F)), _full((_F, _F)),
                _full((1, _F)),
            ],
            out_specs=[blk_a, blk_a],
            out_shape=[jax.ShapeDtypeStruct((_NA, _F), f32),
                       jax.ShapeDtypeStruct((_NA, _F), f32)],
        )(h, osc, msg_W1[nxt], m_b1[nxt], m_w2[nxt], m_b2[nxt])

    def c1b_call(i, hn, big_h):
        return pl.pallas_call(
            functools.partial(_c1b_body, i == 0),
            grid=(_NAT,),
            in_specs=[
                blk_a, blk_i3,
                pl.BlockSpec((_AT, _BPAD), lambda a: (a, 0)),
                _full((_NCG, 1)), _full((_NCG, _F)),
                _full((_F, _F)), _full((1, _F)), _full((_F, _F)),
                _full((1, _F)), _full((_BPAD, _F)),
            ],
            out_specs=_full((_NCG, _F)),
            out_shape=jax.ShapeDtypeStruct((_NCG, _F), f32),
            scratch_shapes=[pltpu.VMEM((_NCG, _F), f32),
                            pltpu.VMEM((_NCG, _F), f32)],
        )(hn, map3, b_ii, cnt, big_h,
          cg_W1[i], c_b1[i], c_w2[i], c_b2[i], c_wd[i])

    big_h = jnp.zeros((_NCG, _F), f32)
    for i in range(3):
        osc = _scb(phi, w_list[i], src, dst)
        if i > 0:
            big_h = c1b_call(i - 1, h, big_h)
        hn, phi = c1a_call(i, h, osc)
        h = hn
    big_h = c1b_call(2, h, big_h)

    return (big_h, h)
